# baseline (speedup 1.0000x reference)
import functools

import jax
import jax.numpy as jnp
from jax import lax
from jax.experimental import pallas as pl
from jax.experimental.pallas import tpu as pltpu
from jax.experimental.pallas import tpu_sc as plsc

NC = 2
NS = 16
NW = NC * NS
L = 16

N_NODES = 100000
E_EDGES = 3200000
B = 128

CH = 2048
NFULL = E_EDGES // CH
TAIL = E_EDGES - NFULL * CH
JMAX = (NFULL + NW - 1) // NW
PAIRS = JMAX // 2
UNR = 8
GPC = CH // (UNR * L)

MAPW = 100096
NONE2 = 0xFFFF

CAP = 4096
MYCAP = 4096
BCAP = 1024
BPT = B // NW

_mesh = plsc.VectorSubcoreMesh(core_axis_name="c", subcore_axis_name="s")


def _iota():
    return lax.iota(jnp.int32, L)


def _append(buf, off_ref, keys, m, cap):
    m32 = m.astype(jnp.int32)
    cs = plsc.cumsum(m32)
    pc = plsc.all_reduce_population_count(m)
    off = off_ref[...]
    pos = jnp.minimum(off + (cs - m32), cap + L - 1)
    plsc.store_scatter(buf, [pos], keys, mask=m)
    off_ref[...] = jnp.minimum(off + pc, cap)


def _append2(buf, off_ref, k1, m1, k2, m2, cap):
    n1 = m1.astype(jnp.int32)
    n2 = m2.astype(jnp.int32)
    c1 = plsc.cumsum(n1)
    c2 = plsc.cumsum(n2)
    p1 = plsc.all_reduce_population_count(m1)
    p2 = plsc.all_reduce_population_count(m2)
    off = off_ref[...]
    pos1 = jnp.minimum(off + (c1 - n1), cap + L - 1)
    pos2 = jnp.minimum(off + p1 + (c2 - n2), cap + L - 1)
    plsc.store_scatter(buf, [pos1], k1, mask=m1)
    plsc.store_scatter(buf, [pos2], k2, mask=m2)
    off_ref[...] = jnp.minimum(off + p1 + p2, cap)


@functools.partial(
    pl.kernel,
    out_type=(
        jax.ShapeDtypeStruct((NW * CAP,), jnp.int32),
        jax.ShapeDtypeStruct((NW * 8,), jnp.int32),
    ),
    mesh=_mesh,
    compiler_params=pltpu.CompilerParams(needs_layout_passes=False),
    scratch_types=[
        pltpu.VMEM((MAPW,), jnp.int32),
        pltpu.VMEM((B,), jnp.int32),
        pltpu.VMEM((2, CH), jnp.int32),
        pltpu.VMEM((2, CH), jnp.int32),
        pltpu.VMEM((CAP + L,), jnp.int32),
        pltpu.VMEM((L,), jnp.int32),
        pltpu.VMEM((L,), jnp.int32),
        pltpu.SemaphoreType.DMA,
        pltpu.SemaphoreType.DMA,
    ],
)
def _scan(ei_hbm, sidx_hbm, hits_hbm, counts_hbm,
          map_v, sidx_v, eb0, eb1, hitbuf, off_v, both_v, se0, se1):
    wid = lax.axis_index("s") * NC + lax.axis_index("c")
    iota = _iota()

    fill = jnp.full((L,), NONE2, jnp.int32)

    def memset(g, _):
        base = g * (8 * L)
        for u in range(8):
            map_v[pl.ds(base + u * L, L)] = fill
        return 0

    lax.fori_loop(0, MAPW // (8 * L), memset, 0)

    pltpu.sync_copy(sidx_hbm, sidx_v)
    off_v[...] = jnp.zeros((L,), jnp.int32)

    for t in range(B // L):
        sv = sidx_v[pl.ds(t * L, L)]
        slots = iota + (t * L)
        delta = slots - 255
        plsc.addupdate_scatter(map_v, [sv], delta)
        dv = sv + 1
        plsc.addupdate_scatter(map_v, [dv], delta * 256, mask=dv < N_NODES)


    def start(j, buf, sem):
        k = wid + NW * j

        @pl.when(k < NFULL)
        def _():
            pltpu.make_async_copy(ei_hbm.at[:, pl.ds(k * CH, CH)], buf, sem).start()

    def wait(j, buf, sem):
        k = wid + NW * j

        @pl.when(k < NFULL)
        def _():
            pltpu.make_async_copy(ei_hbm.at[:, pl.ds(0, CH)], buf, sem).wait()

    def process(buf, ngroups):
        both_v[...] = jnp.zeros((L,), jnp.int32)

        def vec(o, _):
            base = o * (UNR * L)
            codes = [plsc.load_gather(map_v, [buf[0, pl.ds(base + u * L, L)]])
                     for u in range(UNR)]
            lo = codes[0]
            for u in range(1, UNR):
                lo = jnp.minimum(lo, codes[u])

            @pl.when(jnp.any(lo != NONE2))
            def _():
                bothg = None
                for u in range(UNR):
                    code = codes[u]
                    cv = buf[1, pl.ds(base + u * L, L)]
                    s = code & 255
                    d = code >> 8
                    sm = s != 255
                    many = code != NONE2
                    slot = jnp.where(sm, s, d)
                    tag = jnp.where(sm, 0, 1)
                    key = (slot << 18) | (cv * 2) | tag
                    _append(hitbuf, off_v, key, many, CAP)
                    bu = sm & (d != 255)
                    bothg = bu if bothg is None else (bothg | bu)
                both_v[...] = both_v[...] | bothg.astype(jnp.int32)

            return 0

        lax.fori_loop(0, ngroups, vec, 0)

        @pl.when(jnp.any(both_v[...] != 0))
        def _():
            def rescue(o, _):
                base = o * (UNR * L)
                for u in range(UNR):
                    code = plsc.load_gather(
                        map_v, [buf[0, pl.ds(base + u * L, L)]])
                    cv = buf[1, pl.ds(base + u * L, L)]
                    d = code >> 8
                    both = ((code & 255) != 255) & (d != 255)
                    _append(hitbuf, off_v,
                            (d << 18) | (cv * 2) | 1, both, CAP)
                return 0

            lax.fori_loop(0, ngroups, rescue, 0)

    def process_if(j, buf, ngroups):
        @pl.when(wid + NW * j < NFULL)
        def _():
            process(buf, ngroups)

    start(0, eb0, se0)

    def pair(g, _):
        start(2 * g + 1, eb1, se1)
        wait(2 * g, eb0, se0)
        process_if(2 * g, eb0, GPC)
        start(2 * g + 2, eb0, se0)
        wait(2 * g + 1, eb1, se1)
        process_if(2 * g + 1, eb1, GPC)
        return 0

    lax.fori_loop(0, PAIRS, pair, 0)
    wait(JMAX - 1, eb0, se0)
    process_if(JMAX - 1, eb0, GPC)

    @pl.when(wid == NW - 1)
    def _():
        pltpu.make_async_copy(ei_hbm.at[:, pl.ds(NFULL * CH, TAIL)],
                              eb1.at[:, pl.ds(0, TAIL)], se1).start()
        pltpu.make_async_copy(ei_hbm.at[:, pl.ds(NFULL * CH, TAIL)],
                              eb1.at[:, pl.ds(0, TAIL)], se1).wait()
        process(eb1, TAIL // (UNR * L))

    cnt = jnp.max(off_v[...])
    nfl = lax.div(cnt + 255, 256)

    def flush(ch, _):
        pltpu.sync_copy(hitbuf.at[pl.ds(ch * 256, 256)],
                        hits_hbm.at[pl.ds(wid * CAP + ch * 256, 256)])
        return 0

    lax.fori_loop(0, nfl, flush, 0)
    pltpu.sync_copy(off_v.at[pl.ds(0, 8)], counts_hbm.at[pl.ds(wid * 8, 8)])


@functools.partial(
    pl.kernel,
    out_type=jax.ShapeDtypeStruct((NW, L), jnp.float32),
    mesh=_mesh,
    compiler_params=pltpu.CompilerParams(needs_layout_passes=False),
    scratch_types=[
        pltpu.VMEM((NW * 8,), jnp.int32),
        pltpu.VMEM((NW * 512,), jnp.int32),
        pltpu.VMEM((256,), jnp.int32),
        pltpu.VMEM((MYCAP + L,), jnp.int32),
        pltpu.VMEM((BCAP + L,), jnp.int32),
        pltpu.VMEM((BCAP + L,), jnp.int32),
        pltpu.VMEM((BCAP + L,), jnp.int32),
        pltpu.VMEM((BCAP + L,), jnp.int32),
        pltpu.VMEM((L,), jnp.int32),
        pltpu.VMEM((L,), jnp.int32),
        pltpu.VMEM((L,), jnp.int32),
        pltpu.VMEM((L,), jnp.int32),
        pltpu.VMEM((L,), jnp.int32),
        pltpu.VMEM((L,), jnp.float32),
        pltpu.SemaphoreType.DMA,
    ],
)
def _join(hits_hbm, counts_hbm, out_hbm,
          counts_v, scanall, scanbuf, mylist, bl0, bl1, bl2, bl3,
          moff_v, bo0, bo1, bo2, bo3, outv, spf):
    wid = lax.axis_index("s") * NC + lax.axis_index("c")
    iota = _iota()
    lo = wid * BPT
    bls = (bl0, bl1, bl2, bl3)
    bos = (bo0, bo1, bo2, bo3)

    for s in range(NW):
        pltpu.make_async_copy(hits_hbm.at[pl.ds(s * CAP, 512)],
                              scanall.at[pl.ds(s * 512, 512)], spf).start()
    pltpu.sync_copy(counts_hbm, counts_v)
    moff_v[...] = jnp.zeros((L,), jnp.int32)
    for t in range(BPT):
        bos[t][...] = jnp.zeros((L,), jnp.int32)
    for s in range(NW):
        pltpu.make_async_copy(hits_hbm.at[pl.ds(s * CAP, 512)],
                              scanall.at[pl.ds(s * 512, 512)], spf).wait()

    def per_src(s, _):
        cnt = jnp.minimum(
            jnp.max(plsc.load_gather(counts_v, [jnp.full((L,), s * 8, jnp.int32)])),
            CAP)
        head = jnp.minimum(cnt, 512)

        def per_vec(i, _):
            kv1 = scanall[pl.ds(s * 512 + i * (2 * L), L)]
            kv2 = scanall[pl.ds(s * 512 + i * (2 * L) + L, L)]
            gpos = i * (2 * L) + iota
            b1 = kv1 >> 18
            b2 = kv2 >> 18
            m1 = (gpos < head) & (b1 >= lo) & (b1 < lo + BPT)
            m2 = (gpos + L < head) & (b2 >= lo) & (b2 < lo + BPT)
            _append2(mylist, moff_v, kv1, m1, kv2, m2, MYCAP)
            return 0

        lax.fori_loop(0, lax.div(head + 2 * L - 1, 2 * L), per_vec, 0)

        @pl.when(cnt > 512)
        def _():
            nch = lax.div(cnt - 512 + 255, 256)

            def per_ch(ch, _):
                pltpu.sync_copy(
                    hits_hbm.at[pl.ds(s * CAP + 512 + ch * 256, 256)], scanbuf)

                def tail_vec(i, _):
                    kv = scanbuf[pl.ds(i * L, L)]
                    gpos = 512 + ch * 256 + i * L + iota
                    b = kv >> 18
                    m = (gpos < cnt) & (b >= lo) & (b < lo + BPT)
                    _append(mylist, moff_v, kv, m, MYCAP)
                    return 0

                lax.fori_loop(0, 256 // L, tail_vec, 0)
                return 0

            lax.fori_loop(0, nch, per_ch, 0)

        return 0

    lax.fori_loop(0, NW, per_src, 0)

    mycnt = jnp.max(moff_v[...])

    def split(i, _):
        kv = mylist[pl.ds(i * L, L)]
        valid = (i * L + iota) < mycnt
        b = kv >> 18
        for t in range(BPT):
            _append(bls[t], bos[t], kv, valid & (b == lo + t), BCAP)

        return 0

    lax.fori_loop(0, lax.div(mycnt + L - 1, L), split, 0)

    acc_cn = jnp.zeros((L,), jnp.float32)
    acc_un = jnp.zeros((L,), jnp.float32)
    for t in range(BPT):
        blt = bls[t]
        ln = jnp.minimum(jnp.max(bos[t][...]), BCAP)
        nj = lax.div(ln + L - 1, L)

        def outer(i, carry, blt=blt, ln=ln, nj=nj):
            cnv, uni = carry
            ki = plsc.load_gather(blt, [jnp.full((L,), i, jnp.int32)])
            ci = ki >> 1
            isrc = (ki & 1) == 0

            def inner(j, c2, blt=blt, ci=ci, isrc=isrc, ln=ln, i=i):
                cnv2, seen = c2
                kv = blt[pl.ds(j * L, L)]
                posj = j * L + iota
                same = ((kv >> 1) == ci) & (posj < ln)
                cnm = same & ((kv & 1) == 1) & isrc
                seen2 = seen | (same & (posj < i)).astype(jnp.int32)
                return (cnv2 + cnm.astype(jnp.int32), seen2)

            cnv, seen = lax.fori_loop(0, nj, inner,
                                      (cnv, jnp.zeros((L,), jnp.int32)))
            uni = uni + jnp.where(jnp.max(seen) > 0, 0, 1)
            return (cnv, uni)

        cnv, uni = lax.fori_loop(0, ln, outer,
                                 (jnp.zeros((L,), jnp.int32), 0))
        cn = jnp.sum(cnv).astype(jnp.float32)
        un = uni.astype(jnp.float32)
        acc_cn = jnp.where(iota == t, jnp.full((L,), cn), acc_cn)
        acc_un = jnp.where(iota == t, jnp.full((L,), un), acc_un)

    outv[...] = jnp.where(acc_un > 0.0,
                          acc_cn / jnp.maximum(acc_un, 1.0),
                          jnp.zeros((L,), jnp.float32))
    pltpu.sync_copy(outv, out_hbm.at[wid])


def kernel(z, edge_index, batch):
    del z
    src_idx = jnp.sum(
        batch[None, :] < jnp.arange(B, dtype=batch.dtype)[:, None],
        axis=1, dtype=jnp.int32)
    hits, counts = _scan(edge_index, src_idx)
    out = _join(hits, counts)
    return out[:, :BPT].reshape(-1)


# speedup vs baseline: 1.0620x; 1.0620x over previous
import functools

import jax
import jax.numpy as jnp
from jax import lax
from jax.experimental import pallas as pl
from jax.experimental.pallas import tpu as pltpu
from jax.experimental.pallas import tpu_sc as plsc

NC = 2
NS = 16
NW = NC * NS
L = 16

N_NODES = 100000
E_EDGES = 3200000
B = 128

CH = 2048
NFULL = E_EDGES // CH
TAIL = E_EDGES - NFULL * CH
JMAX = (NFULL + NW - 1) // NW
PAIRS = JMAX // 2
UNR = 8
GPC = CH // (UNR * L)

MAPW = 100096
NONE2 = 0xFFFF

CAP = 4096
MYCAP = 4096
BCAP = 1024
BPT = B // NW

_mesh = plsc.VectorSubcoreMesh(core_axis_name="c", subcore_axis_name="s")


def _iota():
    return lax.iota(jnp.int32, L)


def _append(buf, off_ref, keys, m, cap):
    m32 = m.astype(jnp.int32)
    cs = plsc.cumsum(m32)
    pc = plsc.all_reduce_population_count(m)
    off = off_ref[...]
    pos = jnp.minimum(off + (cs - m32), cap + L - 1)
    plsc.store_scatter(buf, [pos], keys, mask=m)
    off_ref[...] = jnp.minimum(off + pc, cap)


def _append2(buf, off_ref, k1, m1, k2, m2, cap):
    n1 = m1.astype(jnp.int32)
    n2 = m2.astype(jnp.int32)
    c1 = plsc.cumsum(n1)
    c2 = plsc.cumsum(n2)
    p1 = plsc.all_reduce_population_count(m1)
    p2 = plsc.all_reduce_population_count(m2)
    off = off_ref[...]
    pos1 = jnp.minimum(off + (c1 - n1), cap + L - 1)
    pos2 = jnp.minimum(off + p1 + (c2 - n2), cap + L - 1)
    plsc.store_scatter(buf, [pos1], k1, mask=m1)
    plsc.store_scatter(buf, [pos2], k2, mask=m2)
    off_ref[...] = jnp.minimum(off + p1 + p2, cap)


@functools.partial(
    pl.kernel,
    out_type=(
        jax.ShapeDtypeStruct((NW * CAP,), jnp.int32),
        jax.ShapeDtypeStruct((NW * 8,), jnp.int32),
    ),
    mesh=_mesh,
    compiler_params=pltpu.CompilerParams(needs_layout_passes=False),
    scratch_types=[
        pltpu.VMEM((MAPW,), jnp.int32),
        pltpu.VMEM((B,), jnp.int32),
        pltpu.VMEM((2, CH), jnp.int32),
        pltpu.VMEM((2, CH), jnp.int32),
        pltpu.VMEM((CAP + L,), jnp.int32),
        pltpu.VMEM((L,), jnp.int32),
        pltpu.VMEM((L,), jnp.int32),
        pltpu.SemaphoreType.DMA,
        pltpu.SemaphoreType.DMA,
    ],
)
def _scan(ei_hbm, sidx_hbm, hits_hbm, counts_hbm,
          map_v, sidx_v, eb0, eb1, hitbuf, off_v, both_v, se0, se1):
    wid = lax.axis_index("s") * NC + lax.axis_index("c")
    iota = _iota()

    fill = jnp.full((L,), NONE2, jnp.int32)

    def memset(g, _):
        base = g * (8 * L)
        for u in range(8):
            map_v[pl.ds(base + u * L, L)] = fill
        return 0

    lax.fori_loop(0, MAPW // (8 * L), memset, 0)

    pltpu.sync_copy(sidx_hbm, sidx_v)
    off_v[...] = jnp.zeros((L,), jnp.int32)

    for t in range(B // L):
        sv = sidx_v[pl.ds(t * L, L)]
        slots = iota + (t * L)
        delta = slots - 255
        plsc.addupdate_scatter(map_v, [sv], delta)
        dv = sv + 1
        plsc.addupdate_scatter(map_v, [dv], delta * 256, mask=dv < N_NODES)


    def start(j, buf, sem):
        k = wid + NW * j

        @pl.when(k < NFULL)
        def _():
            pltpu.make_async_copy(ei_hbm.at[:, pl.ds(k * CH, CH)], buf, sem).start()

    def wait(j, buf, sem):
        k = wid + NW * j

        @pl.when(k < NFULL)
        def _():
            pltpu.make_async_copy(ei_hbm.at[:, pl.ds(0, CH)], buf, sem).wait()

    def process(buf, ngroups):
        both_v[...] = jnp.zeros((L,), jnp.int32)

        def vec(o, _):
            base = o * (UNR * L)
            codes = [plsc.load_gather(map_v, [buf[0, pl.ds(base + u * L, L)]])
                     for u in range(UNR)]
            lo = codes[0]
            for u in range(1, UNR):
                lo = jnp.minimum(lo, codes[u])

            @pl.when(jnp.any(lo != NONE2))
            def _():
                bothg = None
                for u in range(UNR):
                    code = codes[u]
                    cv = buf[1, pl.ds(base + u * L, L)]
                    s = code & 255
                    d = code >> 8
                    sm = s != 255
                    many = code != NONE2
                    slot = jnp.where(sm, s, d)
                    tag = jnp.where(sm, 0, 1)
                    key = (slot << 18) | (cv * 2) | tag
                    _append(hitbuf, off_v, key, many, CAP)
                    bu = sm & (d != 255)
                    bothg = bu if bothg is None else (bothg | bu)
                both_v[...] = both_v[...] | bothg.astype(jnp.int32)

            return 0

        lax.fori_loop(0, ngroups, vec, 0)

        @pl.when(jnp.any(both_v[...] != 0))
        def _():
            def rescue(o, _):
                base = o * (UNR * L)
                for u in range(UNR):
                    code = plsc.load_gather(
                        map_v, [buf[0, pl.ds(base + u * L, L)]])
                    cv = buf[1, pl.ds(base + u * L, L)]
                    d = code >> 8
                    both = ((code & 255) != 255) & (d != 255)
                    _append(hitbuf, off_v,
                            (d << 18) | (cv * 2) | 1, both, CAP)
                return 0

            lax.fori_loop(0, ngroups, rescue, 0)

    def process_if(j, buf, ngroups):
        @pl.when(wid + NW * j < NFULL)
        def _():
            process(buf, ngroups)

    start(0, eb0, se0)

    def pair(g, _):
        start(2 * g + 1, eb1, se1)
        wait(2 * g, eb0, se0)
        process_if(2 * g, eb0, GPC)
        start(2 * g + 2, eb0, se0)
        wait(2 * g + 1, eb1, se1)
        process_if(2 * g + 1, eb1, GPC)
        return 0

    lax.fori_loop(0, PAIRS, pair, 0)
    wait(JMAX - 1, eb0, se0)
    process_if(JMAX - 1, eb0, GPC)

    @pl.when(wid == NW - 1)
    def _():
        pltpu.make_async_copy(ei_hbm.at[:, pl.ds(NFULL * CH, TAIL)],
                              eb1.at[:, pl.ds(0, TAIL)], se1).start()
        pltpu.make_async_copy(ei_hbm.at[:, pl.ds(NFULL * CH, TAIL)],
                              eb1.at[:, pl.ds(0, TAIL)], se1).wait()
        process(eb1, TAIL // (UNR * L))

    cnt = jnp.max(off_v[...])
    nfl = lax.div(cnt + 255, 256)

    def flush(ch, _):
        pltpu.sync_copy(hitbuf.at[pl.ds(ch * 256, 256)],
                        hits_hbm.at[pl.ds(wid * CAP + ch * 256, 256)])
        return 0

    lax.fori_loop(0, nfl, flush, 0)
    pltpu.sync_copy(off_v.at[pl.ds(0, 8)], counts_hbm.at[pl.ds(wid * 8, 8)])


@functools.partial(
    pl.kernel,
    out_type=jax.ShapeDtypeStruct((NW, L), jnp.float32),
    mesh=_mesh,
    compiler_params=pltpu.CompilerParams(needs_layout_passes=False),
    scratch_types=[
        pltpu.VMEM((NW * 8,), jnp.int32),
        pltpu.VMEM((NW * 512,), jnp.int32),
        pltpu.VMEM((256,), jnp.int32),
        pltpu.VMEM((MYCAP + L,), jnp.int32),
        pltpu.VMEM((BCAP + L,), jnp.int32),
        pltpu.VMEM((BCAP + L,), jnp.int32),
        pltpu.VMEM((BCAP + L,), jnp.int32),
        pltpu.VMEM((BCAP + L,), jnp.int32),
        pltpu.VMEM((L,), jnp.int32),
        pltpu.VMEM((L,), jnp.int32),
        pltpu.VMEM((L,), jnp.int32),
        pltpu.VMEM((L,), jnp.int32),
        pltpu.VMEM((L,), jnp.int32),
        pltpu.VMEM((L,), jnp.float32),
        pltpu.SemaphoreType.DMA,
    ],
)
def _join(hits_hbm, counts_hbm, out_hbm,
          counts_v, scanall, scanbuf, mylist, bl0, bl1, bl2, bl3,
          moff_v, bo0, bo1, bo2, bo3, outv, spf):
    wid = lax.axis_index("s") * NC + lax.axis_index("c")
    iota = _iota()
    lo = wid * BPT
    bls = (bl0, bl1, bl2, bl3)
    bos = (bo0, bo1, bo2, bo3)

    for s in range(NW):
        pltpu.make_async_copy(hits_hbm.at[pl.ds(s * CAP, 512)],
                              scanall.at[pl.ds(s * 512, 512)], spf).start()
    pltpu.sync_copy(counts_hbm, counts_v)
    moff_v[...] = jnp.zeros((L,), jnp.int32)
    for t in range(BPT):
        bos[t][...] = jnp.zeros((L,), jnp.int32)
    for s in range(NW):
        pltpu.make_async_copy(hits_hbm.at[pl.ds(s * CAP, 512)],
                              scanall.at[pl.ds(s * 512, 512)], spf).wait()

    def per_src(s, _):
        cnt = jnp.minimum(
            jnp.max(plsc.load_gather(counts_v, [jnp.full((L,), s * 8, jnp.int32)])),
            CAP)
        head = jnp.minimum(cnt, 512)

        def per_vec(i, _):
            kv1 = scanall[pl.ds(s * 512 + i * (2 * L), L)]
            kv2 = scanall[pl.ds(s * 512 + i * (2 * L) + L, L)]
            gpos = i * (2 * L) + iota
            b1 = kv1 >> 18
            b2 = kv2 >> 18
            m1 = (gpos < head) & (b1 >= lo) & (b1 < lo + BPT)
            m2 = (gpos + L < head) & (b2 >= lo) & (b2 < lo + BPT)
            _append2(mylist, moff_v, kv1, m1, kv2, m2, MYCAP)
            return 0

        lax.fori_loop(0, lax.div(head + 2 * L - 1, 2 * L), per_vec, 0)

        @pl.when(cnt > 512)
        def _():
            nch = lax.div(cnt - 512 + 255, 256)

            def per_ch(ch, _):
                pltpu.sync_copy(
                    hits_hbm.at[pl.ds(s * CAP + 512 + ch * 256, 256)], scanbuf)

                def tail_vec(i, _):
                    kv = scanbuf[pl.ds(i * L, L)]
                    gpos = 512 + ch * 256 + i * L + iota
                    b = kv >> 18
                    m = (gpos < cnt) & (b >= lo) & (b < lo + BPT)
                    _append(mylist, moff_v, kv, m, MYCAP)
                    return 0

                lax.fori_loop(0, 256 // L, tail_vec, 0)
                return 0

            lax.fori_loop(0, nch, per_ch, 0)

        return 0

    lax.fori_loop(0, NW, per_src, 0)

    mycnt = jnp.max(moff_v[...])

    def split(i, _):
        kv = mylist[pl.ds(i * L, L)]
        valid = (i * L + iota) < mycnt
        b = kv >> 18
        for t in range(BPT):
            _append(bls[t], bos[t], kv, valid & (b == lo + t), BCAP)

        return 0

    lax.fori_loop(0, lax.div(mycnt + L - 1, L), split, 0)

    acc_cn = jnp.zeros((L,), jnp.float32)
    acc_un = jnp.zeros((L,), jnp.float32)
    for t in range(BPT):
        blt = bls[t]
        ln = jnp.minimum(jnp.max(bos[t][...]), BCAP)
        nj = lax.div(ln + L - 1, L)

        def outer(i, carry, blt=blt, ln=ln, nj=nj):
            cnv, uni = carry
            ki = plsc.load_gather(blt, [jnp.full((L,), i, jnp.int32)])
            ci = ki >> 1
            isrc = (ki & 1) == 0

            def inner(j, c2, blt=blt, ci=ci, isrc=isrc, ln=ln, i=i):
                cnv2, seen = c2
                kv = blt[pl.ds(j * L, L)]
                posj = j * L + iota
                same = ((kv >> 1) == ci) & (posj < ln)
                cnm = same & ((kv & 1) == 1) & isrc
                seen2 = seen | (same & (posj < i)).astype(jnp.int32)
                return (cnv2 + cnm.astype(jnp.int32), seen2)

            cnv, seen = lax.fori_loop(0, nj, inner,
                                      (cnv, jnp.zeros((L,), jnp.int32)))
            uni = uni + jnp.where(jnp.max(seen) > 0, 0, 1)
            return (cnv, uni)

        cnv, uni = lax.fori_loop(0, ln, outer,
                                 (jnp.zeros((L,), jnp.int32), 0))
        cn = jnp.sum(cnv).astype(jnp.float32)
        un = uni.astype(jnp.float32)
        acc_cn = jnp.where(iota == t, jnp.full((L,), cn), acc_cn)
        acc_un = jnp.where(iota == t, jnp.full((L,), un), acc_un)

    outv[...] = jnp.where(acc_un > 0.0,
                          acc_cn / jnp.maximum(acc_un, 1.0),
                          jnp.zeros((L,), jnp.float32))
    pltpu.sync_copy(outv, out_hbm.at[wid])


def kernel(z, edge_index, batch):
    del z
    src_idx = jnp.searchsorted(
        batch, jnp.arange(B, dtype=batch.dtype), side="left").astype(jnp.int32)
    hits, counts = _scan(edge_index, src_idx)
    out = _join(hits, counts)
    return out[:, :BPT].reshape(-1)
